# baseline (speedup 1.0000x reference)
import jax
import jax.numpy as jnp
from jax import lax
from jax.experimental import pallas as pl
from jax.experimental.pallas import tpu as pltpu
from jax.experimental.pallas import tpu_sc as plsc

_L = 16
_NT = 16


def _rsqrt16(v):
    i = lax.bitcast_convert_type(v, jnp.int32)
    i = jnp.int32(0x5F3759DF) - lax.shift_right_arithmetic(i, jnp.int32(1))
    y = lax.bitcast_convert_type(i, jnp.float32)
    half, threehalf = jnp.float32(0.5), jnp.float32(1.5)
    for _ in range(3):
        y = y * (threehalf - half * v * y * y)
    return y


def kernel(x, edge_index, k=2):
    n, dfeat = x.shape
    e = edge_index.shape[1]
    assert dfeat % (2 * _L) == 0
    C = dfeat // 2
    QC = C // _L
    NP = ((n + 2047) // 2048) * 2048
    RT = NP // _NT
    NCH = RT // 128
    assert e % 128 == 0
    EB = e // 128
    NBT = EB // _NT
    REM = EB % _NT
    NBUF = 8
    HALF = NBUF // 2
    f32 = jnp.float32

    xs = jnp.stack([x[:, :C], x[:, C:]])
    xs = jnp.pad(xs, ((0, 0), (0, NP - n), (0, 0)))
    sdb = jnp.stack([edge_index[0].reshape(EB, 128),
                     edge_index[1].reshape(EB, 128)], axis=1)
    karr = jnp.full((_L,), k, jnp.int32)

    def body(*refs):
        (x_hbm, sd_hbm, k_hbm,
         out_hbm, y_hbm,
         acc_sh, deg_sh, stage_sh,
         zbuf, z128, ones128) = refs[:11]
        p = 11
        rows_b = refs[p:p + NBUF]; p += NBUF
        sdi_b = refs[p:p + NBUF]; p += NBUF
        gsem = refs[p:p + NBUF]; p += NBUF
        ssem = refs[p:p + NBUF]; p += NBUF
        dsem = refs[p:p + NBUF]; p += NBUF
        stagev, degv, svec, dloc, kbuf = refs[p:p + 5]
        accv = rows_b[0]
        yv = rows_b[1]
        c = lax.axis_index("c")
        s = lax.axis_index("s")
        r0 = s * RT
        T = NBT + jnp.where(s < REM, 1, 0)
        g0 = s * NBT + jnp.minimum(s, REM)
        NRB = (T // NBUF) * NBUF
        zv = jnp.zeros((_L,), f32)
        ov = jnp.ones((_L,), f32)

        def fill_const(row, _):
            for q in range(QC):
                zbuf[row, pl.ds(q * _L, _L)] = zv
            return 0
        lax.fori_loop(0, 128, fill_const, 0)
        for i in range(128 // _L):
            z128[pl.ds(i * _L, _L)] = zv
            ones128[pl.ds(i * _L, _L)] = ov
        pltpu.sync_copy(k_hbm, kbuf)
        kk = kbuf[:][0]

        def dsplat(r):
            return plsc.load_gather(dloc, [jnp.full((_L,), r, jnp.int32)])

        def sd_load(gb, j):
            return pltpu.make_async_copy(sd_hbm.at[gb], sdi_b[j], dsem[j])

        scope_a = jax.named_scope("ph_a_zero_stats"); scope_a.__enter__()
        for j in range(NCH):
            rr = r0 + j * 128
            pltpu.sync_copy(zbuf, acc_sh.at[pl.ds(rr, 128)])
            pltpu.sync_copy(z128, deg_sh.at[pl.ds(rr, 128)])

        def stat_rowb(buf):
            def f(i, cy):
                ss, qq = cy
                for r in range(4):
                    row = i * 4 + r
                    ssn, qqn = [], []
                    for q in range(QC):
                        v = buf[row, pl.ds(q * _L, _L)]
                        ssn.append(ss[q] + v)
                        qqn.append(qq[q] + v * v)
                    ss, qq = tuple(ssn), tuple(qqn)
                return (ss, qq)
            return f

        cy = (tuple([zv] * QC), tuple([zv] * QC))
        for j in range(2):
            pltpu.async_copy(x_hbm.at[c].at[pl.ds(r0 + j * 128, 128)],
                             rows_b[j], gsem[j])
        for j in range(NCH):
            pltpu.make_async_copy(x_hbm.at[c].at[pl.ds(r0 + j * 128, 128)],
                                  rows_b[j % 2], gsem[j % 2]).wait()
            cy = lax.fori_loop(0, 32, stat_rowb(rows_b[j % 2]), cy)
            if j + 2 < NCH:
                pltpu.async_copy(
                    x_hbm.at[c].at[pl.ds(r0 + (j + 2) * 128, 128)],
                    rows_b[j % 2], gsem[j % 2])
        ssum_p, qsum_p = cy
        for q in range(QC):
            svec[pl.ds(q * _L, _L)] = ssum_p[q]
        pltpu.sync_copy(svec, stage_sh.at[0].at[s])
        for q in range(QC):
            svec[pl.ds(q * _L, _L)] = qsum_p[q]
        pltpu.sync_copy(svec, stage_sh.at[1].at[s])
        plsc.subcore_barrier()
        scope_a.__exit__(None, None, None)
        scope_d = jax.named_scope("ph_deg"); scope_d.__enter__()

        for j in range(NBUF):
            sd_load(g0 + j, j).start()

        def deg_round(i, _):
            for h in range(2):
                Jh = range(h * HALF, (h + 1) * HALF)
                for j in Jh:
                    sd_load(g0 + i * NBUF + j, j).wait()
                for j in Jh:
                    pltpu.async_copy(ones128, deg_sh.at[sdi_b[j].at[0]],
                                     ssem[j], add=True)
                for j in Jh:
                    pltpu.make_async_copy(ones128,
                                          deg_sh.at[sdi_b[j].at[0]],
                                          ssem[j]).wait()
                for j in Jh:
                    b = i * NBUF + j

                    @pl.when(b + NBUF < NRB)
                    def _(j=j, b=b):
                        sd_load(g0 + b + NBUF, j).start()
            return 0
        lax.fori_loop(0, T // NBUF, deg_round, 0)

        def deg_tail(b, _):
            pltpu.sync_copy(sd_hbm.at[g0 + b], sdi_b[0])
            pltpu.sync_copy(ones128, deg_sh.at[sdi_b[0].at[0]], add=True)
            return 0
        lax.fori_loop(NRB, T, deg_tail, 0)
        plsc.subcore_barrier()
        scope_d.__exit__(None, None, None)
        scope_b = jax.named_scope("ph_b_c_y0"); scope_b.__enter__()

        def red(t, cy2):
            return tuple(cy2[q] + stagev[t, pl.ds(q * _L, _L)]
                         for q in range(QC))
        pltpu.sync_copy(stage_sh.at[0], stagev)
        ssum = lax.fori_loop(0, _NT, red, tuple([zv] * QC))
        pltpu.sync_copy(stage_sh.at[1], stagev)
        qsum = lax.fori_loop(0, _NT, red, tuple([zv] * QC))
        rn = f32(1.0 / n)
        rn1 = f32(1.0 / (n - 1))
        mean, rstd = [], []
        for q in range(QC):
            m = ssum[q] * rn
            mean.append(m)
            var = (qsum[q] - f32(n) * m * m) * rn1
            rstd.append(jnp.where(var > 0, _rsqrt16(var), jnp.float32(1.0)))

        for j in range(NCH):
            pltpu.sync_copy(deg_sh.at[pl.ds(r0 + j * 128, 128)], degv)
            for i in range(128 // _L):
                v = degv[pl.ds(i * _L, _L)]
                dloc[pl.ds(j * 128 + i * _L, _L)] = _rsqrt16(v + f32(2.0))

        for j in range(2):
            pltpu.async_copy(x_hbm.at[c].at[pl.ds(r0 + j * 128, 128)],
                             rows_b[j], gsem[j])
        for j in range(NCH):
            rr = r0 + j * 128
            xb, ob = rows_b[j % 2], rows_b[2 + j % 2]
            pltpu.make_async_copy(x_hbm.at[c].at[pl.ds(rr, 128)], xb,
                                  gsem[j % 2]).wait()
            if j >= 2:
                pltpu.make_async_copy(
                    ob, y_hbm.at[c].at[pl.ds(r0 + (j - 2) * 128, 128)],
                    ssem[j % 2]).wait()

            def c_row(i, _, j=j, xb=xb, ob=ob):
                for r in range(4):
                    row = i * 4 + r
                    sc = dsplat(j * 128 + row)
                    for q in range(QC):
                        v = ((xb[row, pl.ds(q * _L, _L)] - mean[q])
                             * rstd[q] * sc)
                        ob[row, pl.ds(q * _L, _L)] = v
                return 0
            lax.fori_loop(0, 32, c_row, 0)
            pltpu.async_copy(ob, y_hbm.at[c].at[pl.ds(rr, 128)], ssem[j % 2])
            if j + 2 < NCH:
                pltpu.async_copy(
                    x_hbm.at[c].at[pl.ds(r0 + (j + 2) * 128, 128)], xb,
                    gsem[j % 2])
        for j in range(NCH - 2, NCH):
            pltpu.make_async_copy(rows_b[2 + j % 2],
                                  y_hbm.at[c].at[pl.ds(r0 + j * 128, 128)],
                                  ssem[j % 2]).wait()
        plsc.subcore_barrier()
        scope_b.__exit__(None, None, None)

        def one_iter(it, _):
            scope_e = jax.named_scope("ph_edge"); scope_e.__enter__()
            for j in range(NBUF):
                sd_load(g0 + j, j).start()
            for j in range(NBUF):
                sd_load(g0 + j, j).wait()
                pltpu.async_copy(y_hbm.at[c].at[sdi_b[j].at[1]], rows_b[j],
                                 gsem[j])

            def edge_round(i, _2):
                for h in range(2):
                    Jh = range(h * HALF, (h + 1) * HALF)
                    for j in Jh:
                        pltpu.make_async_copy(
                            y_hbm.at[c].at[sdi_b[j].at[1]], rows_b[j],
                            gsem[j]).wait()
                    for j in Jh:
                        pltpu.async_copy(rows_b[j],
                                         acc_sh.at[sdi_b[j].at[0]],
                                         ssem[j], add=True)
                    for j in Jh:
                        pltpu.make_async_copy(
                            rows_b[j], acc_sh.at[sdi_b[j].at[0]],
                            ssem[j]).wait()
                    for j in Jh:
                        b = i * NBUF + j

                        @pl.when(b + NBUF < NRB)
                        def _(j=j, b=b):
                            sd_load(g0 + b + NBUF, j).start()
                    for j in Jh:
                        b = i * NBUF + j

                        @pl.when(b + NBUF < NRB)
                        def _(j=j, b=b):
                            sd_load(g0 + b + NBUF, j).wait()
                            pltpu.async_copy(
                                y_hbm.at[c].at[sdi_b[j].at[1]], rows_b[j],
                                gsem[j])
                return 0
            lax.fori_loop(0, T // NBUF, edge_round, 0)

            def edge_tail(b, _2):
                pltpu.sync_copy(sd_hbm.at[g0 + b], sdi_b[0])
                pltpu.sync_copy(y_hbm.at[c].at[sdi_b[0].at[1]], rows_b[0])
                pltpu.sync_copy(rows_b[0], acc_sh.at[sdi_b[0].at[0]],
                                add=True)
                return 0
            lax.fori_loop(NRB, T, edge_tail, 0)
            plsc.subcore_barrier()
            scope_e.__exit__(None, None, None)
            scope_dr = jax.named_scope("ph_drain"); scope_dr.__enter__()
            for j in range(2):
                rr = r0 + j * 128
                pltpu.async_copy(acc_sh.at[pl.ds(rr, 128)], rows_b[j],
                                 gsem[j])
                pltpu.async_copy(y_hbm.at[c].at[pl.ds(rr, 128)],
                                 rows_b[2 + j], gsem[2 + j])
            for j in range(NCH):
                rr = r0 + j * 128
                ab, yb, ob = rows_b[j % 2], rows_b[2 + j % 2], rows_b[4 + j % 2]
                pltpu.make_async_copy(acc_sh.at[pl.ds(rr, 128)], ab,
                                      gsem[j % 2]).wait()
                pltpu.make_async_copy(y_hbm.at[c].at[pl.ds(rr, 128)], yb,
                                      gsem[2 + j % 2]).wait()
                if j >= 2:
                    rp = r0 + (j - 2) * 128
                    pltpu.make_async_copy(
                        ob, y_hbm.at[c].at[pl.ds(rp, 128)],
                        ssem[j % 2]).wait()
                    pltpu.make_async_copy(zbuf, acc_sh.at[pl.ds(rp, 128)],
                                          dsem[j % 2]).wait()

                def d_row(i, _3, j=j, ab=ab, yb=yb, ob=ob):
                    for r in range(4):
                        row = i * 4 + r
                        sc = dsplat(j * 128 + row)
                        s2 = sc * sc
                        for q in range(QC):
                            v = s2 * (ab[row, pl.ds(q * _L, _L)]
                                      + f32(2.0) * yb[row, pl.ds(q * _L, _L)])
                            ob[row, pl.ds(q * _L, _L)] = v
                    return 0
                lax.fori_loop(0, 32, d_row, 0)
                pltpu.async_copy(ob, y_hbm.at[c].at[pl.ds(rr, 128)],
                                 ssem[j % 2])
                pltpu.async_copy(zbuf, acc_sh.at[pl.ds(rr, 128)],
                                 dsem[j % 2])
                if j + 2 < NCH:
                    rn2 = r0 + (j + 2) * 128
                    pltpu.async_copy(acc_sh.at[pl.ds(rn2, 128)], ab,
                                     gsem[j % 2])
                    pltpu.async_copy(y_hbm.at[c].at[pl.ds(rn2, 128)], yb,
                                     gsem[2 + j % 2])
            for j in range(NCH - 2, NCH):
                rr = r0 + j * 128
                pltpu.make_async_copy(rows_b[4 + j % 2],
                                      y_hbm.at[c].at[pl.ds(rr, 128)],
                                      ssem[j % 2]).wait()
                pltpu.make_async_copy(zbuf, acc_sh.at[pl.ds(rr, 128)],
                                      dsem[j % 2]).wait()
            plsc.subcore_barrier()
            scope_dr.__exit__(None, None, None)
            return 0
        lax.fori_loop(0, kk, one_iter, 0)

        scope_u = jax.named_scope("ph_unscale"); scope_u.__enter__()
        for j in range(2):
            pltpu.async_copy(y_hbm.at[c].at[pl.ds(r0 + j * 128, 128)],
                             rows_b[j], gsem[j])
        for j in range(NCH):
            rr = r0 + j * 128
            yb, ob = rows_b[j % 2], rows_b[2 + j % 2]
            pltpu.make_async_copy(y_hbm.at[c].at[pl.ds(rr, 128)], yb,
                                  gsem[j % 2]).wait()
            if j >= 2:
                pltpu.make_async_copy(
                    ob, out_hbm.at[c].at[pl.ds(r0 + (j - 2) * 128, 128)],
                    ssem[j % 2]).wait()

            def e_row(i, _, j=j, yb=yb, ob=ob):
                for r in range(4):
                    row = i * 4 + r
                    sc = dsplat(j * 128 + row)
                    for q in range(QC):
                        ob[row, pl.ds(q * _L, _L)] = (
                            yb[row, pl.ds(q * _L, _L)] / sc)
                return 0
            lax.fori_loop(0, 32, e_row, 0)
            pltpu.async_copy(ob, out_hbm.at[c].at[pl.ds(rr, 128)],
                             ssem[j % 2])
            if j + 2 < NCH:
                pltpu.async_copy(
                    y_hbm.at[c].at[pl.ds(r0 + (j + 2) * 128, 128)], yb,
                    gsem[j % 2])
        for j in range(NCH - 2, NCH):
            pltpu.make_async_copy(rows_b[2 + j % 2],
                                  out_hbm.at[c].at[pl.ds(r0 + j * 128, 128)],
                                  ssem[j % 2]).wait()
        scope_u.__exit__(None, None, None)

    f = pl.kernel(
        body,
        out_type=(jax.ShapeDtypeStruct((2, NP, C), f32),
                  jax.ShapeDtypeStruct((2, NP, C), f32)),
        mesh=plsc.VectorSubcoreMesh(core_axis_name="c", subcore_axis_name="s"),
        compiler_params=pltpu.CompilerParams(needs_layout_passes=False,
                                             use_tc_tiling_on_sc=False),
        scratch_types=[
            pltpu.VMEM_SHARED((NP, C), f32),
            pltpu.VMEM_SHARED((NP,), f32),
            pltpu.VMEM_SHARED((2, _NT, C), f32),
            pltpu.VMEM((128, C), f32),
            pltpu.VMEM((128,), f32),
            pltpu.VMEM((128,), f32),
            *([pltpu.VMEM((128, C), f32)] * NBUF),
            *([pltpu.VMEM((2, 128), jnp.int32)] * NBUF),
            *([pltpu.SemaphoreType.DMA] * (3 * NBUF)),
            pltpu.VMEM((_NT, C), f32),
            pltpu.VMEM((128,), f32),
            pltpu.VMEM((C,), f32),
            pltpu.VMEM((RT,), f32),
            pltpu.VMEM((_L,), jnp.int32),
        ],
    )
    out2, _ = f(xs, sdb, karr)
    return jnp.concatenate([out2[0, :n], out2[1, :n]], axis=1)


# speedup vs baseline: 1.0121x; 1.0121x over previous
import jax
import jax.numpy as jnp
from jax import lax
from jax.experimental import pallas as pl
from jax.experimental.pallas import tpu as pltpu
from jax.experimental.pallas import tpu_sc as plsc

_L = 16
_NT = 16


def _rsqrt16(v):
    i = lax.bitcast_convert_type(v, jnp.int32)
    i = jnp.int32(0x5F3759DF) - lax.shift_right_arithmetic(i, jnp.int32(1))
    y = lax.bitcast_convert_type(i, jnp.float32)
    half, threehalf = jnp.float32(0.5), jnp.float32(1.5)
    for _ in range(3):
        y = y * (threehalf - half * v * y * y)
    return y


def kernel(x, edge_index, k=2):
    n, dfeat = x.shape
    e = edge_index.shape[1]
    assert dfeat % (2 * _L) == 0
    C = dfeat // 2
    QC = C // _L
    NP = ((n + 2047) // 2048) * 2048
    RT = NP // _NT
    NCH = RT // 128
    assert e % 128 == 0
    EB = e // 128
    NBT = EB // _NT
    REM = EB % _NT
    NBUF = 6
    HALF = NBUF // 2
    f32 = jnp.float32

    xs = jnp.stack([x[:, :C], x[:, C:]])
    xs = jnp.pad(xs, ((0, 0), (0, NP - n), (0, 0)))
    sdb = jnp.stack([edge_index[0].reshape(EB, 128),
                     edge_index[1].reshape(EB, 128)], axis=1)
    karr = jnp.full((_L,), k, jnp.int32)

    def body(*refs):
        (x_hbm, sd_hbm, k_hbm,
         out_hbm, y_hbm,
         acc_sh, deg_sh, stage_sh,
         zbuf, z128, ones128) = refs[:11]
        p = 11
        rows_b = refs[p:p + NBUF]; p += NBUF
        sdi_b = refs[p:p + NBUF]; p += NBUF
        gsem = refs[p:p + NBUF]; p += NBUF
        ssem = refs[p:p + NBUF]; p += NBUF
        dsem = refs[p:p + NBUF]; p += NBUF
        stagev, degv, svec, dloc, kbuf = refs[p:p + 5]
        accv = rows_b[0]
        yv = rows_b[1]
        c = lax.axis_index("c")
        s = lax.axis_index("s")
        r0 = s * RT
        T = NBT + jnp.where(s < REM, 1, 0)
        g0 = s * NBT + jnp.minimum(s, REM)
        NRB = (T // NBUF) * NBUF
        zv = jnp.zeros((_L,), f32)
        ov = jnp.ones((_L,), f32)

        def fill_const(row, _):
            for q in range(QC):
                zbuf[row, pl.ds(q * _L, _L)] = zv
            return 0
        lax.fori_loop(0, 128, fill_const, 0)
        for i in range(128 // _L):
            z128[pl.ds(i * _L, _L)] = zv
            ones128[pl.ds(i * _L, _L)] = ov
        pltpu.sync_copy(k_hbm, kbuf)
        kk = kbuf[:][0]

        def dsplat(r):
            return plsc.load_gather(dloc, [jnp.full((_L,), r, jnp.int32)])

        def sd_load(gb, j):
            return pltpu.make_async_copy(sd_hbm.at[gb], sdi_b[j], dsem[j])

        scope_a = jax.named_scope("ph_a_zero_stats"); scope_a.__enter__()
        for j in range(NCH):
            rr = r0 + j * 128
            pltpu.sync_copy(zbuf, acc_sh.at[pl.ds(rr, 128)])
            pltpu.sync_copy(z128, deg_sh.at[pl.ds(rr, 128)])

        def stat_rowb(buf):
            def f(row, cy):
                ss, qq = cy
                ssn, qqn = [], []
                for q in range(QC):
                    v = buf[row, pl.ds(q * _L, _L)]
                    ssn.append(ss[q] + v)
                    qqn.append(qq[q] + v * v)
                return (tuple(ssn), tuple(qqn))
            return f

        cy = (tuple([zv] * QC), tuple([zv] * QC))
        for j in range(2):
            pltpu.async_copy(x_hbm.at[c].at[pl.ds(r0 + j * 128, 128)],
                             rows_b[j], gsem[j])
        for j in range(NCH):
            pltpu.make_async_copy(x_hbm.at[c].at[pl.ds(r0 + j * 128, 128)],
                                  rows_b[j % 2], gsem[j % 2]).wait()
            cy = lax.fori_loop(0, 128, stat_rowb(rows_b[j % 2]), cy)
            if j + 2 < NCH:
                pltpu.async_copy(
                    x_hbm.at[c].at[pl.ds(r0 + (j + 2) * 128, 128)],
                    rows_b[j % 2], gsem[j % 2])
        ssum_p, qsum_p = cy
        for q in range(QC):
            svec[pl.ds(q * _L, _L)] = ssum_p[q]
        pltpu.sync_copy(svec, stage_sh.at[0].at[s])
        for q in range(QC):
            svec[pl.ds(q * _L, _L)] = qsum_p[q]
        pltpu.sync_copy(svec, stage_sh.at[1].at[s])
        plsc.subcore_barrier()
        scope_a.__exit__(None, None, None)
        scope_d = jax.named_scope("ph_deg"); scope_d.__enter__()

        for j in range(NBUF):
            sd_load(g0 + j, j).start()

        def deg_round(i, _):
            for h in range(2):
                Jh = range(h * HALF, (h + 1) * HALF)
                for j in Jh:
                    sd_load(g0 + i * NBUF + j, j).wait()
                for j in Jh:
                    pltpu.async_copy(ones128, deg_sh.at[sdi_b[j].at[0]],
                                     ssem[j], add=True)
                for j in Jh:
                    pltpu.make_async_copy(ones128,
                                          deg_sh.at[sdi_b[j].at[0]],
                                          ssem[j]).wait()
                for j in Jh:
                    b = i * NBUF + j

                    @pl.when(b + NBUF < NRB)
                    def _(j=j, b=b):
                        sd_load(g0 + b + NBUF, j).start()
            return 0
        lax.fori_loop(0, T // NBUF, deg_round, 0)

        def deg_tail(b, _):
            pltpu.sync_copy(sd_hbm.at[g0 + b], sdi_b[0])
            pltpu.sync_copy(ones128, deg_sh.at[sdi_b[0].at[0]], add=True)
            return 0
        lax.fori_loop(NRB, T, deg_tail, 0)
        plsc.subcore_barrier()
        scope_d.__exit__(None, None, None)
        scope_b = jax.named_scope("ph_b_c_y0"); scope_b.__enter__()

        def red(t, cy2):
            return tuple(cy2[q] + stagev[t, pl.ds(q * _L, _L)]
                         for q in range(QC))
        pltpu.sync_copy(stage_sh.at[0], stagev)
        ssum = lax.fori_loop(0, _NT, red, tuple([zv] * QC))
        pltpu.sync_copy(stage_sh.at[1], stagev)
        qsum = lax.fori_loop(0, _NT, red, tuple([zv] * QC))
        rn = f32(1.0 / n)
        rn1 = f32(1.0 / (n - 1))
        mean, rstd = [], []
        for q in range(QC):
            m = ssum[q] * rn
            mean.append(m)
            var = (qsum[q] - f32(n) * m * m) * rn1
            rstd.append(jnp.where(var > 0, _rsqrt16(var), jnp.float32(1.0)))

        for j in range(NCH):
            pltpu.sync_copy(deg_sh.at[pl.ds(r0 + j * 128, 128)], degv)
            for i in range(128 // _L):
                v = degv[pl.ds(i * _L, _L)]
                dloc[pl.ds(j * 128 + i * _L, _L)] = _rsqrt16(v + f32(2.0))

        for j in range(2):
            pltpu.async_copy(x_hbm.at[c].at[pl.ds(r0 + j * 128, 128)],
                             rows_b[j], gsem[j])
        for j in range(NCH):
            rr = r0 + j * 128
            xb, ob = rows_b[j % 2], rows_b[2 + j % 2]
            pltpu.make_async_copy(x_hbm.at[c].at[pl.ds(rr, 128)], xb,
                                  gsem[j % 2]).wait()
            if j >= 2:
                pltpu.make_async_copy(
                    ob, y_hbm.at[c].at[pl.ds(r0 + (j - 2) * 128, 128)],
                    ssem[j % 2]).wait()

            def c_row(row, _, j=j, xb=xb, ob=ob):
                sc = dsplat(j * 128 + row)
                for q in range(QC):
                    v = (xb[row, pl.ds(q * _L, _L)] - mean[q]) * rstd[q] * sc
                    ob[row, pl.ds(q * _L, _L)] = v
                return 0
            lax.fori_loop(0, 128, c_row, 0)
            pltpu.async_copy(ob, y_hbm.at[c].at[pl.ds(rr, 128)], ssem[j % 2])
            if j + 2 < NCH:
                pltpu.async_copy(
                    x_hbm.at[c].at[pl.ds(r0 + (j + 2) * 128, 128)], xb,
                    gsem[j % 2])
        for j in range(NCH - 2, NCH):
            pltpu.make_async_copy(rows_b[2 + j % 2],
                                  y_hbm.at[c].at[pl.ds(r0 + j * 128, 128)],
                                  ssem[j % 2]).wait()
        plsc.subcore_barrier()
        scope_b.__exit__(None, None, None)

        def one_iter(it, _):
            scope_e = jax.named_scope("ph_edge"); scope_e.__enter__()
            for j in range(NBUF):
                sd_load(g0 + j, j).start()
            for j in range(NBUF):
                sd_load(g0 + j, j).wait()
                pltpu.async_copy(y_hbm.at[c].at[sdi_b[j].at[1]], rows_b[j],
                                 gsem[j])

            def edge_round(i, _2):
                for h in range(2):
                    Jh = range(h * HALF, (h + 1) * HALF)
                    for j in Jh:
                        pltpu.make_async_copy(
                            y_hbm.at[c].at[sdi_b[j].at[1]], rows_b[j],
                            gsem[j]).wait()
                    for j in Jh:
                        pltpu.async_copy(rows_b[j],
                                         acc_sh.at[sdi_b[j].at[0]],
                                         ssem[j], add=True)
                    for j in Jh:
                        pltpu.make_async_copy(
                            rows_b[j], acc_sh.at[sdi_b[j].at[0]],
                            ssem[j]).wait()
                    for j in Jh:
                        b = i * NBUF + j

                        @pl.when(b + NBUF < NRB)
                        def _(j=j, b=b):
                            sd_load(g0 + b + NBUF, j).start()
                    for j in Jh:
                        b = i * NBUF + j

                        @pl.when(b + NBUF < NRB)
                        def _(j=j, b=b):
                            sd_load(g0 + b + NBUF, j).wait()
                            pltpu.async_copy(
                                y_hbm.at[c].at[sdi_b[j].at[1]], rows_b[j],
                                gsem[j])
                return 0
            lax.fori_loop(0, T // NBUF, edge_round, 0)

            def edge_tail(b, _2):
                pltpu.sync_copy(sd_hbm.at[g0 + b], sdi_b[0])
                pltpu.sync_copy(y_hbm.at[c].at[sdi_b[0].at[1]], rows_b[0])
                pltpu.sync_copy(rows_b[0], acc_sh.at[sdi_b[0].at[0]],
                                add=True)
                return 0
            lax.fori_loop(NRB, T, edge_tail, 0)
            plsc.subcore_barrier()
            scope_e.__exit__(None, None, None)
            scope_dr = jax.named_scope("ph_drain"); scope_dr.__enter__()
            for j in range(2):
                rr = r0 + j * 128
                pltpu.async_copy(acc_sh.at[pl.ds(rr, 128)], rows_b[j],
                                 gsem[j])
                pltpu.async_copy(y_hbm.at[c].at[pl.ds(rr, 128)],
                                 rows_b[2 + j], gsem[2 + j])
            for j in range(NCH):
                rr = r0 + j * 128
                ab, yb, ob = rows_b[j % 2], rows_b[2 + j % 2], rows_b[4 + j % 2]
                pltpu.make_async_copy(acc_sh.at[pl.ds(rr, 128)], ab,
                                      gsem[j % 2]).wait()
                pltpu.make_async_copy(y_hbm.at[c].at[pl.ds(rr, 128)], yb,
                                      gsem[2 + j % 2]).wait()
                if j >= 2:
                    rp = r0 + (j - 2) * 128
                    pltpu.make_async_copy(
                        ob, y_hbm.at[c].at[pl.ds(rp, 128)],
                        ssem[j % 2]).wait()
                    pltpu.make_async_copy(zbuf, acc_sh.at[pl.ds(rp, 128)],
                                          dsem[j % 2]).wait()

                def d_row(row, _3, j=j, ab=ab, yb=yb, ob=ob):
                    sc = dsplat(j * 128 + row)
                    s2 = sc * sc
                    for q in range(QC):
                        v = s2 * (ab[row, pl.ds(q * _L, _L)]
                                  + f32(2.0) * yb[row, pl.ds(q * _L, _L)])
                        ob[row, pl.ds(q * _L, _L)] = v
                    return 0
                lax.fori_loop(0, 128, d_row, 0)
                pltpu.async_copy(ob, y_hbm.at[c].at[pl.ds(rr, 128)],
                                 ssem[j % 2])
                pltpu.async_copy(zbuf, acc_sh.at[pl.ds(rr, 128)],
                                 dsem[j % 2])
                if j + 2 < NCH:
                    rn2 = r0 + (j + 2) * 128
                    pltpu.async_copy(acc_sh.at[pl.ds(rn2, 128)], ab,
                                     gsem[j % 2])
                    pltpu.async_copy(y_hbm.at[c].at[pl.ds(rn2, 128)], yb,
                                     gsem[2 + j % 2])
            for j in range(NCH - 2, NCH):
                rr = r0 + j * 128
                pltpu.make_async_copy(rows_b[4 + j % 2],
                                      y_hbm.at[c].at[pl.ds(rr, 128)],
                                      ssem[j % 2]).wait()
                pltpu.make_async_copy(zbuf, acc_sh.at[pl.ds(rr, 128)],
                                      dsem[j % 2]).wait()
            plsc.subcore_barrier()
            scope_dr.__exit__(None, None, None)
            return 0
        lax.fori_loop(0, kk, one_iter, 0)

        scope_u = jax.named_scope("ph_unscale"); scope_u.__enter__()
        for j in range(2):
            pltpu.async_copy(y_hbm.at[c].at[pl.ds(r0 + j * 128, 128)],
                             rows_b[j], gsem[j])
        for j in range(NCH):
            rr = r0 + j * 128
            yb, ob = rows_b[j % 2], rows_b[2 + j % 2]
            pltpu.make_async_copy(y_hbm.at[c].at[pl.ds(rr, 128)], yb,
                                  gsem[j % 2]).wait()
            if j >= 2:
                pltpu.make_async_copy(
                    ob, out_hbm.at[c].at[pl.ds(r0 + (j - 2) * 128, 128)],
                    ssem[j % 2]).wait()

            def e_row(row, _, j=j, yb=yb, ob=ob):
                sc = dsplat(j * 128 + row)
                for q in range(QC):
                    ob[row, pl.ds(q * _L, _L)] = (
                        yb[row, pl.ds(q * _L, _L)] / sc)
                return 0
            lax.fori_loop(0, 128, e_row, 0)
            pltpu.async_copy(ob, out_hbm.at[c].at[pl.ds(rr, 128)],
                             ssem[j % 2])
            if j + 2 < NCH:
                pltpu.async_copy(
                    y_hbm.at[c].at[pl.ds(r0 + (j + 2) * 128, 128)], yb,
                    gsem[j % 2])
        for j in range(NCH - 2, NCH):
            pltpu.make_async_copy(rows_b[2 + j % 2],
                                  out_hbm.at[c].at[pl.ds(r0 + j * 128, 128)],
                                  ssem[j % 2]).wait()
        scope_u.__exit__(None, None, None)

    f = pl.kernel(
        body,
        out_type=(jax.ShapeDtypeStruct((2, NP, C), f32),
                  jax.ShapeDtypeStruct((2, NP, C), f32)),
        mesh=plsc.VectorSubcoreMesh(core_axis_name="c", subcore_axis_name="s"),
        compiler_params=pltpu.CompilerParams(needs_layout_passes=False,
                                             use_tc_tiling_on_sc=False),
        scratch_types=[
            pltpu.VMEM_SHARED((NP, C), f32),
            pltpu.VMEM_SHARED((NP,), f32),
            pltpu.VMEM_SHARED((2, _NT, C), f32),
            pltpu.VMEM((128, C), f32),
            pltpu.VMEM((128,), f32),
            pltpu.VMEM((128,), f32),
            *([pltpu.VMEM((128, C), f32)] * NBUF),
            *([pltpu.VMEM((2, 128), jnp.int32)] * NBUF),
            *([pltpu.SemaphoreType.DMA] * (3 * NBUF)),
            pltpu.VMEM((_NT, C), f32),
            pltpu.VMEM((128,), f32),
            pltpu.VMEM((C,), f32),
            pltpu.VMEM((RT,), f32),
            pltpu.VMEM((_L,), jnp.int32),
        ],
    )
    out2, _ = f(xs, sdb, karr)
    return jnp.concatenate([out2[0, :n], out2[1, :n]], axis=1)
